# baseline (speedup 1.0000x reference)
import functools

import jax
import jax.numpy as jnp
from jax import lax
from jax.experimental import pallas as pl
from jax.experimental.pallas import tpu as pltpu
from jax.experimental.pallas import tpu_sc as plsc

_B, _N, _E = 4, 10000, 320000
_D = 128
_DO = 64
_NS = 16
_EPT = _E // _NS
_CHUNK = 80
_NCHUNK = _EPT // _CHUNK
_GROUPS = _CHUNK // 16
_SPT = 640
_WC = 80


def _mm_kernel(x_ref, w_ref, o_ref, *, relu, out_bf16):
    xv = x_ref[...]
    if relu:
        xv = jnp.maximum(xv, 0.0)
    yv = jnp.dot(xv, w_ref[...], preferred_element_type=jnp.float32)
    o_ref[...] = yv.astype(jnp.bfloat16) if out_bf16 else yv


def _mm(x2, w, relu, out_bf16=False):
    m, k = x2.shape
    blk = 2000
    return pl.pallas_call(
        functools.partial(_mm_kernel, relu=relu, out_bf16=out_bf16),
        grid=(m // blk,),
        in_specs=[pl.BlockSpec((blk, k), lambda i: (i, 0)),
                  pl.BlockSpec(w.shape, lambda i: (0, 0))],
        out_specs=pl.BlockSpec((blk, w.shape[1]), lambda i: (i, 0)),
        out_shape=jax.ShapeDtypeStruct(
            (m, w.shape[1]), jnp.bfloat16 if out_bf16 else jnp.float32),
    )(x2, w)


def _pack_rows(hbf):
    st = jnp.stack([hbf[:, :_D // 2], hbf[:, _D // 2:]], axis=-1)
    return st.reshape(hbf.shape[0], _D)


def _mm_bias_kernel(x_ref, w_ref, b_ref, o_ref):
    xv = jnp.maximum(x_ref[...], 0.0)
    o_ref[...] = (jnp.dot(xv, w_ref[...], preferred_element_type=jnp.float32)
                  + b_ref[...])


def _mm_bias(x2, w, b2):
    m, k = x2.shape
    blk = 2000
    return pl.pallas_call(
        _mm_bias_kernel,
        grid=(m // blk,),
        in_specs=[pl.BlockSpec((blk, k), lambda i: (i, 0)),
                  pl.BlockSpec(w.shape, lambda i: (0, 0)),
                  pl.BlockSpec(b2.shape, lambda i: (0, 0))],
        out_specs=pl.BlockSpec((blk, w.shape[1]), lambda i: (i, 0)),
        out_shape=jax.ShapeDtypeStruct((m, w.shape[1]), jnp.float32),
    )(x2, w, b2)


_RING = 4
_STEADY = _NCHUNK - 2


def _spmm_body(h_hbm, ei_hbm, vals_hbm, out_hbm,
               acc, rowb, idxb, dstb, dprv, valb, semg, sems_, seme):
    c = lax.axis_index("c")
    s = lax.axis_index("s")
    ebase = pl.multiple_of(s * _EPT, 8)

    rb = pl.multiple_of(s * _SPT, 8)
    nz = jnp.where(s == _NS - 1, (_N - (_NS - 1) * _SPT) // _WC, _SPT // _WC)

    def edge_start(k, b):
        eoff = ebase + pl.multiple_of(k * _CHUNK, 8)
        pltpu.async_copy(ei_hbm.at[pl.ds(_E + eoff, _CHUNK)], idxb[b], seme[b])
        pltpu.async_copy(ei_hbm.at[pl.ds(eoff, _CHUNK)], dstb[b], seme[b])
        pltpu.async_copy(vals_hbm.at[pl.ds(eoff, _CHUNK)], valb[b], seme[b])

    def edge_wait(b):
        off = pl.ds(ebase, _CHUNK)
        pltpu.make_async_copy(ei_hbm.at[off], idxb[b], seme[b]).wait()
        pltpu.make_async_copy(ei_hbm.at[off], dstb[b], seme[b]).wait()
        pltpu.make_async_copy(vals_hbm.at[off], valb[b], seme[b]).wait()

    def adjust(b, roff):
        for g in range(_GROUPS):
            sl = pl.ds(g * 16, 16)
            idxb[b][sl] = idxb[b][sl] + roff

    def gather_start(b):
        pltpu.async_copy(h_hbm.at[idxb[b]], rowb[b], semg[b])

    def gather_wait(b):
        pltpu.make_async_copy(h_hbm.at[idxb[b]], rowb[b], semg[b]).wait()

    def scale(b):
        @plsc.parallel_loop(0, _GROUPS)
        def _grp(g):
            v16 = valb[b][pl.ds(pl.multiple_of(g * 16, 8), 16)]
            for j in range(16):
                e = g * 16 + j
                bv = jnp.broadcast_to(v16[j], (16,))
                for u in range(_D // 16):
                    sl = pl.ds(u * 16, 16)
                    rowb[b][e, sl] = rowb[b][e, sl] * bv

    def scatter_start(b):
        for g in range(_GROUPS):
            sl = pl.ds(g * 16, 16)
            dprv[b][sl] = dstb[b][sl]
        pltpu.async_copy(rowb[b], acc.at[dprv[b]], sems_[b], add=True)

    def scatter_wait(b):
        pltpu.make_async_copy(rowb[b], acc.at[dprv[b]], sems_[b]).wait()

    for p in range(2):
        roff = (c * 2 + p) * _N

        def _zrow(r, carry):
            for u in range(_D // 16):
                rowb[0][r, pl.ds(u * 16, 16)] = jnp.zeros((16,), jnp.float32)
            return carry
        lax.fori_loop(0, _WC, _zrow, 0)

        def _zcp(z, carry):
            pltpu.async_copy(
                rowb[0], acc.at[pl.ds(pl.multiple_of(rb + z * _WC, 8), _WC)],
                semg[0])
            return carry
        lax.fori_loop(0, nz, _zcp, 0)

        def _zwait(z, carry):
            pltpu.make_async_copy(rowb[0], acc.at[pl.ds(rb, _WC)],
                                  semg[0]).wait()
            return carry
        lax.fori_loop(0, nz, _zwait, 0)
        plsc.subcore_barrier()

        for r in range(_RING):
            edge_start(r, r)
        for r in range(2):
            edge_wait(r)
            adjust(r, roff)
            gather_start(r)

        def _steady(k, r):
            rn = (r + 2) % _RING

            @pl.when(k >= 2)
            def _():
                scatter_wait(rn)

            @pl.when(k < _NCHUNK - 2)
            def _():
                edge_wait(rn)
                adjust(rn, roff)
                gather_start(rn)

            gather_wait(r)
            scale(r)
            scatter_start(r)

            @pl.when(k < _NCHUNK - _RING)
            def _():
                edge_start(k + _RING, r)

        def _outer(k0, carry):
            for j in range(_RING):
                _steady(k0 * _RING + j, j)
            return carry
        lax.fori_loop(0, _STEADY // _RING, _outer, 0)
        for k in (_STEADY, _STEADY + 1):
            _steady(jnp.int32(k), k % _RING)
        scatter_wait(0)
        scatter_wait(1)
        plsc.subcore_barrier()

        def _wcp(z, carry):
            zo = pl.multiple_of(rb + z * _WC, 8)
            pltpu.sync_copy(acc.at[pl.ds(zo, _WC)],
                            out_hbm.at[pl.ds(roff + zo, _WC)])
            return carry
        lax.fori_loop(0, nz, _wcp, 0)
        if p == 0:
            plsc.subcore_barrier()


def _spmm(h, eif, vals):
    f = pl.kernel(
        _spmm_body,
        out_type=jax.ShapeDtypeStruct((_B * _N, _D), jnp.float32),
        mesh=plsc.VectorSubcoreMesh(core_axis_name="c", subcore_axis_name="s"),
        scratch_types=[
            pltpu.VMEM_SHARED((_N, _D), jnp.float32),
            [pltpu.VMEM((_CHUNK, _D), jnp.float32)] * _RING,
            [pltpu.VMEM((_CHUNK,), jnp.int32)] * _RING,
            [pltpu.VMEM((_CHUNK,), jnp.int32)] * _RING,
            [pltpu.VMEM((_CHUNK,), jnp.int32)] * _RING,
            [pltpu.VMEM((_CHUNK,), jnp.float32)] * _RING,
            [pltpu.SemaphoreType.DMA] * _RING,
            [pltpu.SemaphoreType.DMA] * _RING,
            [pltpu.SemaphoreType.DMA] * _RING,
        ],
    )
    return f(h, eif, vals)


def kernel(x, edge_index, adj_values, W1, W2, W_mean, b_mean, W_var, b_var):
    x2 = x.reshape(_B * _N, _D)
    eif = edge_index.reshape(2 * _E)
    wcat = jnp.concatenate([W_mean, W_var], axis=1)
    bcat = jnp.concatenate([b_mean, b_var]).reshape(1, 2 * _DO)

    h0 = _mm(x2, W1, relu=False)
    s0 = _spmm(h0, eif, adj_values)
    h1 = _mm(s0, W2, relu=True)
    s1 = _spmm(h1, eif, adj_values)
    out = _mm_bias(s1, wcat, bcat)
    mean = out[:, :_DO].reshape(_B, _N, _DO)
    log_var = out[:, _DO:].reshape(_B, _N, _DO)
    return mean, log_var


# speedup vs baseline: 1.1539x; 1.1539x over previous
import functools

import jax
import jax.numpy as jnp
from jax import lax
from jax.experimental import pallas as pl
from jax.experimental.pallas import tpu as pltpu
from jax.experimental.pallas import tpu_sc as plsc

_B, _N, _E = 4, 10000, 320000
_D = 128
_DO = 64
_NS = 16
_EPT = _E // _NS
_CHUNK = 80
_NCHUNK = _EPT // _CHUNK
_GROUPS = _CHUNK // 16
_SPT = 640
_WC = 80


def _mm_kernel(x_ref, w_ref, o_ref, *, relu, out_bf16):
    xv = x_ref[...]
    if relu:
        xv = jnp.maximum(xv, 0.0)
    yv = jnp.dot(xv, w_ref[...], preferred_element_type=jnp.float32)
    o_ref[...] = yv.astype(jnp.bfloat16) if out_bf16 else yv


def _mm(x2, w, relu, out_bf16=False):
    m, k = x2.shape
    blk = 2000
    return pl.pallas_call(
        functools.partial(_mm_kernel, relu=relu, out_bf16=out_bf16),
        grid=(m // blk,),
        in_specs=[pl.BlockSpec((blk, k), lambda i: (i, 0)),
                  pl.BlockSpec(w.shape, lambda i: (0, 0))],
        out_specs=pl.BlockSpec((blk, w.shape[1]), lambda i: (i, 0)),
        out_shape=jax.ShapeDtypeStruct(
            (m, w.shape[1]), jnp.bfloat16 if out_bf16 else jnp.float32),
    )(x2, w)


def _pack_rows(hbf):
    st = jnp.stack([hbf[:, :_D // 2], hbf[:, _D // 2:]], axis=-1)
    return st.reshape(hbf.shape[0], _D)


def _mm_bias_kernel(x_ref, w_ref, b_ref, o_ref):
    xv = jnp.maximum(x_ref[...], 0.0)
    o_ref[...] = (jnp.dot(xv, w_ref[...], preferred_element_type=jnp.float32)
                  + b_ref[...])


def _mm_bias(x2, w, b2):
    m, k = x2.shape
    blk = 2000
    return pl.pallas_call(
        _mm_bias_kernel,
        grid=(m // blk,),
        in_specs=[pl.BlockSpec((blk, k), lambda i: (i, 0)),
                  pl.BlockSpec(w.shape, lambda i: (0, 0)),
                  pl.BlockSpec(b2.shape, lambda i: (0, 0))],
        out_specs=pl.BlockSpec((blk, w.shape[1]), lambda i: (i, 0)),
        out_shape=jax.ShapeDtypeStruct((m, w.shape[1]), jnp.float32),
    )(x2, w, b2)


_RING = 4
_STEADY = _NCHUNK - 2


def _spmm_body(h_hbm, ei_hbm, vals_hbm, out_hbm,
               acc, rowb, idxb, dstb, dprv, valb, semg, sems_, seme):
    c = lax.axis_index("c")
    s = lax.axis_index("s")
    ebase = pl.multiple_of(s * _EPT, 8)

    rb = pl.multiple_of(s * _SPT, 8)
    nz = jnp.where(s == _NS - 1, (_N - (_NS - 1) * _SPT) // _WC, _SPT // _WC)

    def edge_start(k, b):
        eoff = ebase + pl.multiple_of(k * _CHUNK, 8)
        pltpu.async_copy(ei_hbm.at[pl.ds(_E + eoff, _CHUNK)], idxb[b], seme[b])
        pltpu.async_copy(ei_hbm.at[pl.ds(eoff, _CHUNK)], dstb[b], seme[b])
        pltpu.async_copy(vals_hbm.at[pl.ds(eoff, _CHUNK)], valb[b], seme[b])

    def edge_wait(b):
        off = pl.ds(ebase, _CHUNK)
        pltpu.make_async_copy(ei_hbm.at[off], idxb[b], seme[b]).wait()
        pltpu.make_async_copy(ei_hbm.at[off], dstb[b], seme[b]).wait()
        pltpu.make_async_copy(vals_hbm.at[off], valb[b], seme[b]).wait()

    def adjust(b, roff):
        for g in range(_GROUPS):
            sl = pl.ds(g * 16, 16)
            idxb[b][sl] = idxb[b][sl] + roff

    def gather_start(b):
        pltpu.async_copy(h_hbm.at[idxb[b]], rowb[b], semg[b])

    def gather_wait(b):
        pltpu.make_async_copy(h_hbm.at[idxb[b]], rowb[b], semg[b]).wait()

    def scale(b):
        def _grp(g, gcarry):
            v16 = valb[b][pl.ds(pl.multiple_of(g * 16, 8), 16)]
            for j in range(16):
                e = g * 16 + j
                bv = jnp.broadcast_to(v16[j], (16,))
                for u in range(_D // 16):
                    sl = pl.ds(u * 16, 16)
                    rowb[b][e, sl] = rowb[b][e, sl] * bv
            return gcarry
        lax.fori_loop(0, _GROUPS, _grp, 0)

    def scatter_start(b):
        for g in range(_GROUPS):
            sl = pl.ds(g * 16, 16)
            dprv[b][sl] = dstb[b][sl]
        pltpu.async_copy(rowb[b], acc.at[dprv[b]], sems_[b], add=True)

    def scatter_wait(b):
        pltpu.make_async_copy(rowb[b], acc.at[dprv[b]], sems_[b]).wait()

    for p in range(2):
        roff = (c * 2 + p) * _N

        for r in range(_RING):
            edge_start(r, r)

        def _zrow(r, carry):
            for u in range(_D // 16):
                rowb[0][r, pl.ds(u * 16, 16)] = jnp.zeros((16,), jnp.float32)
            return carry
        lax.fori_loop(0, _WC, _zrow, 0)

        def _zcp(z, carry):
            pltpu.async_copy(
                rowb[0], acc.at[pl.ds(pl.multiple_of(rb + z * _WC, 8), _WC)],
                semg[0])
            return carry
        lax.fori_loop(0, nz, _zcp, 0)

        def _zwait(z, carry):
            pltpu.make_async_copy(rowb[0], acc.at[pl.ds(rb, _WC)],
                                  semg[0]).wait()
            return carry
        lax.fori_loop(0, nz, _zwait, 0)
        plsc.subcore_barrier()

        for r in range(2):
            edge_wait(r)
            adjust(r, roff)
            gather_start(r)

        def _steady(k, r):
            rn = (r + 2) % _RING

            @pl.when(k >= 2)
            def _():
                scatter_wait(rn)

            @pl.when(k < _NCHUNK - 2)
            def _():
                edge_wait(rn)
                adjust(rn, roff)
                gather_start(rn)

            gather_wait(r)
            scale(r)
            scatter_start(r)

            @pl.when(k < _NCHUNK - _RING)
            def _():
                edge_start(k + _RING, r)

        def _outer(k0, carry):
            for j in range(_RING):
                _steady(k0 * _RING + j, j)
            return carry
        lax.fori_loop(0, _STEADY // _RING, _outer, 0)
        for k in (_STEADY, _STEADY + 1):
            _steady(jnp.int32(k), k % _RING)
        scatter_wait(0)
        scatter_wait(1)
        plsc.subcore_barrier()

        def _wcp(z, carry):
            zo = pl.multiple_of(rb + z * _WC, 8)
            pltpu.async_copy(acc.at[pl.ds(zo, _WC)],
                             out_hbm.at[pl.ds(roff + zo, _WC)], semg[1])
            return carry
        lax.fori_loop(0, nz, _wcp, 0)

        def _wwait(z, carry):
            pltpu.make_async_copy(acc.at[pl.ds(rb, _WC)],
                                  out_hbm.at[pl.ds(roff, _WC)],
                                  semg[1]).wait()
            return carry
        lax.fori_loop(0, nz, _wwait, 0)
        if p == 0:
            plsc.subcore_barrier()


def _spmm(h, eif, vals):
    f = pl.kernel(
        _spmm_body,
        out_type=jax.ShapeDtypeStruct((_B * _N, _D), jnp.float32),
        mesh=plsc.VectorSubcoreMesh(core_axis_name="c", subcore_axis_name="s"),
        scratch_types=[
            pltpu.VMEM_SHARED((_N, _D), jnp.float32),
            [pltpu.VMEM((_CHUNK, _D), jnp.float32)] * _RING,
            [pltpu.VMEM((_CHUNK,), jnp.int32)] * _RING,
            [pltpu.VMEM((_CHUNK,), jnp.int32)] * _RING,
            [pltpu.VMEM((_CHUNK,), jnp.int32)] * _RING,
            [pltpu.VMEM((_CHUNK,), jnp.float32)] * _RING,
            [pltpu.SemaphoreType.DMA] * _RING,
            [pltpu.SemaphoreType.DMA] * _RING,
            [pltpu.SemaphoreType.DMA] * _RING,
        ],
    )
    return f(h, eif, vals)


def kernel(x, edge_index, adj_values, W1, W2, W_mean, b_mean, W_var, b_var):
    x2 = x.reshape(_B * _N, _D)
    eif = edge_index.reshape(2 * _E)
    wcat = jnp.concatenate([W_mean, W_var], axis=1)
    bcat = jnp.concatenate([b_mean, b_var]).reshape(1, 2 * _DO)

    h0 = _mm(x2, W1, relu=False)
    s0 = _spmm(h0, eif, adj_values)
    h1 = _mm(s0, W2, relu=True)
    s1 = _spmm(h1, eif, adj_values)
    out = _mm_bias(s1, wcat, bcat)
    mean = out[:, :_DO].reshape(_B, _N, _DO)
    log_var = out[:, _DO:].reshape(_B, _N, _DO)
    return mean, log_var
